# baseline (speedup 1.0000x reference)
import functools

import jax
import jax.numpy as jnp
import numpy as np
from jax.experimental import pallas as pl

B = 4
SEQ = 1920
PRED = 128
ENC_IN = 7
MARK = 4
D = 512
H = 8
DH = D // H
DFF = 2048
NL = 2
BUCKET = 8
NH = 4
L = SEQ + PRED
BH = B * H
NB = L // BUCKET
TQ = 128
NT = L // TQ
GT = NH * NT

_NEG_BIG = -1e30
_SELF_MASK_VAL = -5e4


def _pe_table():
    pos = np.arange(L)[:, None].astype(np.float64)
    div = np.exp(np.arange(0, D, 2).astype(np.float64) * -(np.log(10000.0) / D))
    pe = np.zeros((L, D))
    pe[:, 0::2] = np.sin(pos * div)
    pe[:, 1::2] = np.cos(pos * div)
    return jnp.asarray(pe, dtype=jnp.float32)


def _embed_body(xcat_ref, w_ref, pe_ref, out_ref):
    xcat = xcat_ref[0]
    w = w_ref[...]
    e0 = jnp.dot(xcat[:, 0:8], w[0:8], preferred_element_type=jnp.float32)
    e1 = jnp.dot(xcat[:, 8:16], w[8:16], preferred_element_type=jnp.float32)
    e2 = jnp.dot(xcat[:, 16:24], w[16:24], preferred_element_type=jnp.float32)
    e3 = jnp.dot(xcat[:, 24:32], w[24:32], preferred_element_type=jnp.float32)
    out_ref[0] = ((e0 + e1) + e2) + pe_ref[...] + e3


def _embed(xcat, w32, pe):
    return pl.pallas_call(
        _embed_body,
        grid=(B,),
        in_specs=[
            pl.BlockSpec((1, L, 32), lambda b: (b, 0, 0)),
            pl.BlockSpec((32, D), lambda b: (0, 0)),
            pl.BlockSpec((L, D), lambda b: (0, 0)),
        ],
        out_specs=pl.BlockSpec((1, L, D), lambda b: (b, 0, 0)),
        out_shape=jax.ShapeDtypeStruct((B, L, D), jnp.float32),
    )(xcat, w32, pe)


def _qkvb_body(h_ref, toqk_ref, tov_ref, rot_ref, qk_ref, v_ref, bk_ref):
    h = h_ref[0]
    qkh = jnp.dot(h, toqk_ref[0], preferred_element_type=jnp.float32)
    v_ref[0] = jnp.dot(h, tov_ref[0], preferred_element_type=jnp.float32)
    qk_ref[0] = qkh
    rot = jnp.dot(qkh, rot_ref[...], preferred_element_type=jnp.float32)
    iota = jax.lax.broadcasted_iota(jnp.int32, (L, 128), 1)
    for r in range(NH):
        s = rot[:, r * 128:(r + 1) * 128]
        mx = jnp.max(s, axis=1)
        mn = jnp.min(s, axis=1)
        idxp = jnp.min(jnp.where(s == mx[:, None], iota, 512), axis=1)
        idxn = jnp.min(jnp.where(s == mn[:, None], iota, 512), axis=1)
        bk_ref[0, r] = jnp.where(mx >= -mn, idxp, 128 + idxn)


def _qkvb(h, toqk, tov, rotflat):
    toqk_r = toqk.reshape(D, H, DH).transpose(1, 0, 2)
    tov_r = tov.reshape(D, H, DH).transpose(1, 0, 2)
    return pl.pallas_call(
        _qkvb_body,
        grid=(B, H),
        in_specs=[
            pl.BlockSpec((1, L, D), lambda b, hh: (b, 0, 0)),
            pl.BlockSpec((1, D, DH), lambda b, hh: (hh, 0, 0)),
            pl.BlockSpec((1, D, DH), lambda b, hh: (hh, 0, 0)),
            pl.BlockSpec((DH, NH * 128), lambda b, hh: (0, 0)),
        ],
        out_specs=[
            pl.BlockSpec((1, L, DH), lambda b, hh: (b * H + hh, 0, 0)),
            pl.BlockSpec((1, L, DH), lambda b, hh: (b * H + hh, 0, 0)),
            pl.BlockSpec((1, NH, L), lambda b, hh: (b * H + hh, 0, 0)),
        ],
        out_shape=[
            jax.ShapeDtypeStruct((BH, L, DH), jnp.float32),
            jax.ShapeDtypeStruct((BH, L, DH), jnp.float32),
            jax.ShapeDtypeStruct((BH, NH, L), jnp.int32),
        ],
    )(h, toqk_r, tov_r, rotflat)


def _attn_body(q_ref, qp_ref, vc_ref, vp_ref, tq_ref, tp_ref, so_ref, lse_ref):
    q = q_ref[0, 0]
    qp = qp_ref[0, 0]
    vc = vc_ref[0, 0]
    vp_blk = vp_ref[0, 0]

    kc = q / (jnp.sqrt(jnp.sum(q * q, axis=1, keepdims=True)) + 1e-9)
    kp_blk = qp / (jnp.sqrt(jnp.sum(qp * qp, axis=1, keepdims=True)) + 1e-9)
    kp = jnp.concatenate([kp_blk[TQ - BUCKET:], kc[:TQ - BUCKET]], axis=0)
    vp = jnp.concatenate([vp_blk[TQ - BUCKET:], vc[:TQ - BUCKET]], axis=0)

    tq = tq_ref[0, 0, 0, 0]
    tp_full = tp_ref[0, 0, 0, 0]
    tkp = jnp.concatenate([tp_full[TQ - BUCKET:], tq[:TQ - BUCKET]], axis=0)

    scale = 1.0 / np.sqrt(DH)
    dn = (((1,), (1,)), ((), ()))
    dots_c = jax.lax.dot_general(q, kc, dn, preferred_element_type=jnp.float32) * scale
    dots_p = jax.lax.dot_general(q, kp, dn, preferred_element_type=jnp.float32) * scale

    ii = jax.lax.broadcasted_iota(jnp.int32, (TQ, TQ), 0) // BUCKET
    jj = jax.lax.broadcasted_iota(jnp.int32, (TQ, TQ), 1) // BUCKET
    band = ii == jj
    self_c = tq[:, None] == tq[None, :]
    self_p = tq[:, None] == tkp[None, :]
    dc = jnp.where(band, jnp.where(self_c, _SELF_MASK_VAL, dots_c), _NEG_BIG)
    dp = jnp.where(band, jnp.where(self_p, _SELF_MASK_VAL, dots_p), _NEG_BIG)

    m = jnp.maximum(jnp.max(dc, axis=1), jnp.max(dp, axis=1))
    ec = jnp.exp(dc - m[:, None])
    ep = jnp.exp(dp - m[:, None])
    s = jnp.sum(ec, axis=1) + jnp.sum(ep, axis=1)
    lse = jnp.log(s) + m
    pc = jnp.exp(dc - lse[:, None])
    pp = jnp.exp(dp - lse[:, None])
    dn2 = (((1,), (0,)), ((), ()))
    so_ref[0, 0] = (jax.lax.dot_general(pc, vc, dn2, preferred_element_type=jnp.float32) +
                    jax.lax.dot_general(pp, vp, dn2, preferred_element_type=jnp.float32))
    lse_ref[0, 0, 0, 0] = lse


def _attention(sqk, sv, st5):
    def prev_map(row, r, t):
        g = (r * NT + t - 1) % GT
        return (row, g // NT, g % NT, 0, 0)

    def prev_map3(row, r, t):
        g = (r * NT + t - 1) % GT
        return (row, g // NT, g % NT)

    grid = (BH, NH, NT)
    so, lse5 = pl.pallas_call(
        _attn_body,
        grid=grid,
        in_specs=[
            pl.BlockSpec((1, 1, TQ, DH), lambda row, r, t: (row, r, t, 0)),
            pl.BlockSpec((1, 1, TQ, DH),
                         lambda row, r, t: prev_map3(row, r, t) + (0,)),
            pl.BlockSpec((1, 1, TQ, DH), lambda row, r, t: (row, r, t, 0)),
            pl.BlockSpec((1, 1, TQ, DH),
                         lambda row, r, t: prev_map3(row, r, t) + (0,)),
            pl.BlockSpec((1, 1, 1, 1, TQ), lambda row, r, t: (row, r, t, 0, 0)),
            pl.BlockSpec((1, 1, 1, 1, TQ), prev_map),
        ],
        out_specs=[
            pl.BlockSpec((1, 1, TQ, DH), lambda row, r, t: (row, r, t, 0)),
            pl.BlockSpec((1, 1, 1, 1, TQ), lambda row, r, t: (row, r, t, 0, 0)),
        ],
        out_shape=[
            jax.ShapeDtypeStruct((BH, NH, L, DH), jnp.float32),
            jax.ShapeDtypeStruct((BH, NH, NT, 1, TQ), jnp.float32),
        ],
    )(sqk, sqk, sv, sv, st5, st5)
    return so, lse5


def _combine_body(o_ref, lg_ref, out_ref):
    lg = lg_ref[0]
    lm = jnp.max(lg, axis=0)
    z = jnp.sum(jnp.exp(lg - lm[None, :]), axis=0)
    p = jnp.exp(lg - (jnp.log(z) + lm)[None, :])
    o = o_ref[0]
    out_ref[0] = jnp.sum(o * p[:, :, None], axis=0)


def _combine(o_u, lg_u):
    return pl.pallas_call(
        _combine_body,
        grid=(BH,),
        in_specs=[
            pl.BlockSpec((1, NH, L, DH), lambda r: (r, 0, 0, 0)),
            pl.BlockSpec((1, NH, L), lambda r: (r, 0, 0)),
        ],
        out_specs=pl.BlockSpec((1, L, DH), lambda r: (r, 0, 0)),
        out_shape=jax.ShapeDtypeStruct((BH, L, DH), jnp.float32),
    )(o_u, lg_u)


def _ln(x, g, b):
    m = x.mean(-1, keepdims=True)
    v = x.var(-1, keepdims=True)
    return (x - m) / jnp.sqrt(v + 1e-5) * g + b


def _tail_body(h_ref, oc_ref, wo_ref, bo_ref, g1_ref, b1_ref, g2_ref, b2_ref,
               w1_ref, bb1_ref, w2_ref, bb2_ref, out_ref):
    h = h_ref[0]
    a = jnp.dot(oc_ref[0], wo_ref[...], preferred_element_type=jnp.float32) + bo_ref[...]
    h1 = _ln(h + a, g1_ref[...], b1_ref[...])
    y = jnp.dot(h1, w1_ref[...], preferred_element_type=jnp.float32) + bb1_ref[...]
    y = jax.nn.gelu(y)
    y = jnp.dot(y, w2_ref[...], preferred_element_type=jnp.float32) + bb2_ref[...]
    out_ref[0] = _ln(h1 + y, g2_ref[...], b2_ref[...])


def _tail(h, oc, p):
    RB = 1024
    full = lambda b, t: (0, 0)
    vec = lambda b, t: (0,)
    return pl.pallas_call(
        _tail_body,
        grid=(B, L // RB),
        in_specs=[
            pl.BlockSpec((1, RB, D), lambda b, t: (b, t, 0)),
            pl.BlockSpec((1, RB, D), lambda b, t: (b, t, 0)),
            pl.BlockSpec((D, D), full),
            pl.BlockSpec((D,), vec),
            pl.BlockSpec((D,), vec),
            pl.BlockSpec((D,), vec),
            pl.BlockSpec((D,), vec),
            pl.BlockSpec((D,), vec),
            pl.BlockSpec((D, DFF), full),
            pl.BlockSpec((DFF,), vec),
            pl.BlockSpec((DFF, D), full),
            pl.BlockSpec((D,), vec),
        ],
        out_specs=pl.BlockSpec((1, RB, D), lambda b, t: (b, t, 0)),
        out_shape=jax.ShapeDtypeStruct((B, L, D), jnp.float32),
    )(h, oc, p['wo'], p['bo'], p['ln1_g'], p['ln1_b'], p['ln2_g'], p['ln2_b'],
      p['w1'], p['b1'], p['w2'], p['b2'])


def _final_body(h_ref, g_ref, b_ref, w_ref, pb_ref, out_ref):
    hn = _ln(h_ref[0], g_ref[...], b_ref[...])
    out_ref[0] = jnp.dot(hn, w_ref[...], preferred_element_type=jnp.float32) + pb_ref[...]


def _final(h, norm_g, norm_b, proj_w_pad, proj_b_pad):
    return pl.pallas_call(
        _final_body,
        grid=(B,),
        in_specs=[
            pl.BlockSpec((1, PRED, D), lambda b: (b, (L // PRED) - 1, 0)),
            pl.BlockSpec((D,), lambda b: (0,)),
            pl.BlockSpec((D,), lambda b: (0,)),
            pl.BlockSpec((D, 128), lambda b: (0, 0)),
            pl.BlockSpec((128,), lambda b: (0,)),
        ],
        out_specs=pl.BlockSpec((1, PRED, 128), lambda b: (b, 0, 0)),
        out_shape=jax.ShapeDtypeStruct((B, PRED, 128), jnp.float32),
    )(h, norm_g, norm_b, proj_w_pad, proj_b_pad)


def _layer(h, p, rotflat):
    qk_all, v_all, buckets = _qkvb(h, p['toqk'], p['tov'], rotflat)

    st = jnp.argsort(buckets, axis=-1, stable=True).astype(jnp.int32)
    undo = jnp.argsort(st, axis=-1).astype(jnp.int32)

    sqk = jnp.take_along_axis(qk_all[:, None], st[..., None], axis=2)
    sv = jnp.take_along_axis(v_all[:, None], st[..., None], axis=2)

    st5 = st.reshape(BH, NH, NT, 1, TQ)
    so, lse5 = _attention(sqk, sv, st5)
    slse = lse5.reshape(BH, NH, L)

    o_u = jnp.take_along_axis(so, undo[..., None], axis=2)
    lg_u = jnp.take_along_axis(slse, undo, axis=2)

    oc = _combine(o_u, lg_u)
    och = oc.reshape(B, H, L, DH).transpose(0, 2, 1, 3).reshape(B, L, D)
    return _tail(h, och, p)


def kernel(x_enc, x_mark_enc, x_dec, x_mark_dec, params):
    x = jnp.concatenate([x_enc, x_dec[:, -PRED:, :]], axis=1)
    xm = jnp.concatenate([x_mark_enc, x_mark_dec[:, -PRED:, :]], axis=1)
    xp = jnp.concatenate([x[:, -1:, :], x, x[:, :1, :]], axis=1)
    z1 = jnp.zeros((B, L, 8 - ENC_IN), jnp.float32)
    z2 = jnp.zeros((B, L, 8 - MARK), jnp.float32)
    xcat = jnp.concatenate(
        [xp[:, 0:L], z1, xp[:, 1:L + 1], z1, xp[:, 2:L + 2], z1, xm, z2],
        axis=-1)
    zw1 = jnp.zeros((8 - ENC_IN, D), jnp.float32)
    zw2 = jnp.zeros((8 - MARK, D), jnp.float32)
    w32 = jnp.concatenate(
        [params['token_w'][0], zw1, params['token_w'][1], zw1,
         params['token_w'][2], zw1, params['temporal_w'], zw2], axis=0)

    h = _embed(xcat, w32, _pe_table())

    rotations = jax.random.normal(jax.random.key(42), (DH, NH, NB // 2),
                                  dtype=jnp.float32)
    rotflat = rotations.reshape(DH, NH * (NB // 2))

    for p in params['layers']:
        h = _layer(h, p, rotflat)

    proj_w_pad = jnp.concatenate(
        [params['proj_w'], jnp.zeros((D, 128 - ENC_IN), jnp.float32)], axis=1)
    proj_b_pad = jnp.concatenate(
        [params['proj_b'], jnp.zeros((128 - ENC_IN,), jnp.float32)], axis=0)
    out = _final(h, params['norm_g'], params['norm_b'], proj_w_pad, proj_b_pad)
    return out[:, :, :ENC_IN]
